# Initial kernel scaffold; baseline (speedup 1.0000x reference)
import jax
import jax.numpy as jnp
from jax.experimental import pallas as pl


def kernel(x, params):
    raise NotImplementedError("write your pallas kernel here")


# speedup vs baseline: 1.2705x; 1.2705x over previous
import numpy as np
import jax
import jax.numpy as jnp
from jax.experimental import pallas as pl

B = 8
T = 12
N = 307
STEPS = 288
IN_EMB = 24
TOD_EMB = 24
DOW_EMB = 24
ADP_EMB = 80
D = IN_EMB + TOD_EMB + DOW_EMB + ADP_EMB
H = 4
HD = D // H
HP = 64
DH = H * HP
FF = 256
NP_ = 320
BT = B * T
GT = 16
RT = GT * T
OUT = 12

_SCALE = 1.0 / np.sqrt(HD)
f32 = jnp.float32
bf16 = jnp.bfloat16

_blockdiag = np.kron(np.eye(GT, dtype=np.float32), np.ones((T, T), np.float32))
_MASK_T = ((1.0 - _blockdiag) * -1e30).astype(np.float32)
_MASK_S = np.zeros((NP_, NP_), np.float32)
_MASK_S[:, N:] = -1e30


def _full_spec(a):
    nd = a.ndim
    return pl.BlockSpec(a.shape, lambda g, _nd=nd: (0,) * _nd)


def _ln(xv, g, b):
    mu = jnp.mean(xv, axis=-1, keepdims=True)
    var = jnp.mean((xv - mu) ** 2, axis=-1, keepdims=True)
    return (xv - mu) * jax.lax.rsqrt(var + 1e-5) * g + b


def _embed_kernel(x_ref, tod_ref, dow_ref, w_ref, b_ref, adp_ref, o_ref):
    xb = x_ref[0]
    h_in = jnp.dot(xb, w_ref[...], preferred_element_type=f32) + b_ref[...]
    t_idx = (xb[:, 1:2] * STEPS).astype(jnp.int32)
    oh_t = (jax.lax.broadcasted_iota(jnp.int32, (NP_, STEPS), 1) == t_idx).astype(f32)
    tod_e = jnp.dot(oh_t, tod_ref[...], preferred_element_type=f32)
    d_idx = xb[:, 2:3].astype(jnp.int32)
    oh_d = (jax.lax.broadcasted_iota(jnp.int32, (NP_, 8), 1) == d_idx).astype(f32)
    dow_e = jnp.dot(oh_d, dow_ref[...], preferred_element_type=f32)
    o_ref[0] = jnp.concatenate([h_in, tod_e, dow_e, adp_ref[0]], axis=1)


def _layer_kernel(x_ref, wqkv_ref, bqkv_ref, wo_ref, bo_ref,
                  g1_ref, be1_ref, w1_ref, b1_ref, w2_ref, b2_ref,
                  g2_ref, be2_ref, mask_ref, o_ref):
    x = x_ref[...]
    qkv = jnp.dot(x.astype(bf16), wqkv_ref[...],
                  preferred_element_type=f32) + bqkv_ref[...]
    mask = mask_ref[...]
    outs = []
    for h in range(H):
        q = qkv[:, HP * h:HP * (h + 1)].astype(bf16)
        k = qkv[:, DH + HP * h:DH + HP * (h + 1)].astype(bf16)
        v = qkv[:, 2 * DH + HP * h:2 * DH + HP * (h + 1)].astype(bf16)
        s = jax.lax.dot_general(q, k, (((1,), (1,)), ((), ())),
                                preferred_element_type=f32) * _SCALE + mask
        m = jnp.max(s, axis=-1, keepdims=True)
        e = jnp.exp(s - m)
        p = (e / jnp.sum(e, axis=-1, keepdims=True)).astype(bf16)
        outs.append(jnp.dot(p, v, preferred_element_type=f32))
    o = jnp.concatenate(outs, axis=1).astype(bf16)
    a = jnp.dot(o, wo_ref[...], preferred_element_type=f32) + bo_ref[...]
    y = _ln(x + a, g1_ref[...], be1_ref[...])
    hm = jnp.dot(y.astype(bf16), w1_ref[...], preferred_element_type=f32) + b1_ref[...]
    hm = jnp.maximum(hm, 0.0).astype(bf16)
    ff = jnp.dot(hm, w2_ref[...], preferred_element_type=f32) + b2_ref[...]
    o_ref[...] = _ln(y + ff, g2_ref[...], be2_ref[...])


def _head_kernel(x_ref, w_ref, b_ref, o_ref):
    o_ref[...] = jnp.dot(x_ref[...].astype(bf16), w_ref[...],
                         preferred_element_type=f32) + b_ref[...]


def _prep_layer(p):
    def headpad_cols(w, b):
        w3 = jnp.pad(w.reshape(D, H, HD), ((0, 0), (0, 0), (0, HP - HD)))
        b2 = jnp.pad(b.reshape(H, HD), ((0, 0), (0, HP - HD)))
        return w3.reshape(D, DH), b2.reshape(DH)

    wq, bq = headpad_cols(p["q"]["w"], p["q"]["b"])
    wk, bk = headpad_cols(p["k"]["w"], p["k"]["b"])
    wv, bv = headpad_cols(p["v"]["w"], p["v"]["b"])
    wqkv = jnp.concatenate([wq, wk, wv], axis=1).astype(bf16)
    bqkv = jnp.concatenate([bq, bk, bv]).reshape(1, 3 * DH)
    wo = jnp.pad(p["o"]["w"].reshape(H, HD, D),
                 ((0, 0), (0, HP - HD), (0, 0))).reshape(DH, D).astype(bf16)
    return [wqkv, bqkv, wo, p["o"]["b"].reshape(1, D),
            p["ln1g"].reshape(1, D), p["ln1b"].reshape(1, D),
            p["ff1"]["w"].astype(bf16), p["ff1"]["b"].reshape(1, FF),
            p["ff2"]["w"].astype(bf16), p["ff2"]["b"].reshape(1, D),
            p["ln2g"].reshape(1, D), p["ln2b"].reshape(1, D)]


def _layer(xf, wts, mask, R):
    mtot = xf.shape[0]
    grid = mtot // R
    in_specs = [pl.BlockSpec((R, D), lambda g: (g, 0))]
    in_specs += [_full_spec(w) for w in wts]
    in_specs.append(_full_spec(mask))
    return pl.pallas_call(
        _layer_kernel,
        grid=(grid,),
        in_specs=in_specs,
        out_specs=pl.BlockSpec((R, D), lambda g: (g, 0)),
        out_shape=jax.ShapeDtypeStruct((mtot, D), f32),
    )(xf, *wts, mask)


def kernel(x, params):
    x2 = jnp.pad(x, ((0, 0), (0, 0), (0, NP_ - N), (0, 0))).reshape(BT, NP_, 3)
    adp = jnp.pad(params["adp"], ((0, 0), (0, NP_ - N), (0, 0)))
    tod_tab = params["tod_tab"]
    dow_tab = jnp.pad(params["dow_tab"], ((0, 1), (0, 0)))
    w_in = params["in_proj"]["w"]
    b_in = params["in_proj"]["b"].reshape(1, IN_EMB)

    emb = pl.pallas_call(
        _embed_kernel,
        grid=(BT,),
        in_specs=[pl.BlockSpec((1, NP_, 3), lambda g: (g, 0, 0)),
                  _full_spec(tod_tab), _full_spec(dow_tab),
                  _full_spec(w_in), _full_spec(b_in),
                  pl.BlockSpec((1, NP_, ADP_EMB), lambda g: (g % T, 0, 0))],
        out_specs=pl.BlockSpec((1, NP_, D), lambda g: (g, 0, 0)),
        out_shape=jax.ShapeDtypeStruct((BT, NP_, D), f32),
    )(x2, tod_tab, dow_tab, w_in, b_in, adp)

    mask_t = jnp.asarray(_MASK_T)
    mask_s = jnp.asarray(_MASK_S)

    h = emb.reshape(B, T, NP_, D).transpose(0, 2, 1, 3).reshape(B * NP_ * T, D)
    for p in params["layers_t"]:
        h = _layer(h, _prep_layer(p), mask_t, RT)

    h = h.reshape(B, NP_, T, D).transpose(0, 2, 1, 3).reshape(BT * NP_, D)
    for p in params["layers_s"]:
        h = _layer(h, _prep_layer(p), mask_s, NP_)

    h = h.reshape(B, T, NP_, D).transpose(0, 2, 1, 3).reshape(B * NP_, T * D)
    w_out = params["out_proj"]["w"]
    b_out = params["out_proj"]["b"].reshape(1, OUT)
    o = pl.pallas_call(
        _head_kernel,
        grid=(B,),
        in_specs=[pl.BlockSpec((NP_, T * D), lambda g: (g, 0)),
                  _full_spec(w_out), _full_spec(b_out)],
        out_specs=pl.BlockSpec((NP_, OUT), lambda g: (g, 0)),
        out_shape=jax.ShapeDtypeStruct((B * NP_, OUT), f32),
    )(h, w_out.astype(bf16), b_out)
    o = o.reshape(B, NP_, OUT)[:, :N].transpose(0, 2, 1)
    return o[..., None]


# speedup vs baseline: 1.3872x; 1.0918x over previous
import numpy as np
import jax
import jax.numpy as jnp
from jax.experimental import pallas as pl

B = 8
T = 12
N = 307
STEPS = 288
IN_EMB = 24
TOD_EMB = 24
DOW_EMB = 24
ADP_EMB = 80
D = IN_EMB + TOD_EMB + DOW_EMB + ADP_EMB
H = 4
HD = D // H
HP = 64
DH = H * HP
FF = 256
NP_ = 320
BT = B * T
GT = 16
RT = GT * T
NGT = NP_ // GT
OUT = 12

_SCALE = 1.0 / np.sqrt(HD)
f32 = jnp.float32
bf16 = jnp.bfloat16

_ii = np.arange(RT)
_MASK_T = np.where((_ii[:, None] - _ii[None, :]) % GT == 0, 0.0,
                   -1e30).astype(np.float32)
_ROWMASK = (np.arange(NP_) < N).astype(np.float32).reshape(NP_, 1)


def _full_spec(a):
    nd = a.ndim
    return pl.BlockSpec(a.shape, lambda g, _nd=nd: (0,) * _nd)


def _ln(xv, g, b):
    mu = jnp.mean(xv, axis=-1, keepdims=True)
    var = jnp.mean((xv - mu) ** 2, axis=-1, keepdims=True)
    return (xv - mu) * jax.lax.rsqrt(var + 1e-5) * g + b


def _embed_kernel(x_ref, tod_ref, dow_ref, w_ref, b_ref, adp_ref, o_ref):
    xb = x_ref[0, 0]
    h_in = jnp.dot(xb, w_ref[...], preferred_element_type=f32) + b_ref[...]
    t_idx = (xb[:, 1:2] * STEPS).astype(jnp.int32)
    oh_t = (jax.lax.broadcasted_iota(jnp.int32, (NP_, STEPS), 1) == t_idx).astype(f32)
    tod_e = jnp.dot(oh_t, tod_ref[...], preferred_element_type=f32)
    d_idx = xb[:, 2:3].astype(jnp.int32)
    oh_d = (jax.lax.broadcasted_iota(jnp.int32, (NP_, 8), 1) == d_idx).astype(f32)
    dow_e = jnp.dot(oh_d, dow_ref[...], preferred_element_type=f32)
    o_ref[0, 0] = jnp.concatenate([h_in, tod_e, dow_e, adp_ref[0]], axis=1)


def _attn_ffn(x, wqkv_ref, bqkv_ref, wo_ref, bo_ref, g1_ref, be1_ref,
              w1_ref, b1_ref, w2_ref, b2_ref, g2_ref, be2_ref,
              mask, rowmask):
    qkv = jnp.dot(x.astype(bf16), wqkv_ref[...],
                  preferred_element_type=f32) + bqkv_ref[...]
    outs = []
    for h in range(H):
        q = qkv[:, HP * h:HP * (h + 1)].astype(bf16)
        k = qkv[:, DH + HP * h:DH + HP * (h + 1)].astype(bf16)
        v = qkv[:, 2 * DH + HP * h:2 * DH + HP * (h + 1)]
        s = jax.lax.dot_general(q, k, (((1,), (1,)), ((), ())),
                                preferred_element_type=f32)
        if mask is not None:
            s = s + mask
            m = jnp.max(s, axis=-1, keepdims=True)
            e = jnp.exp(s - m)
            den = jnp.sum(e, axis=-1, keepdims=True)
        else:
            m = jnp.max(s[:, :N], axis=-1, keepdims=True)
            e = jnp.exp(s - m)
            den = jnp.sum(e[:, :N], axis=-1, keepdims=True)
            v = v * rowmask
        o_h = jnp.dot(e.astype(bf16), v.astype(bf16),
                      preferred_element_type=f32)
        outs.append(o_h * jax.lax.reciprocal(den))
    o = jnp.concatenate(outs, axis=1).astype(bf16)
    a = jnp.dot(o, wo_ref[...], preferred_element_type=f32) + bo_ref[...]
    y = _ln(x + a, g1_ref[...], be1_ref[...])
    hm = jnp.dot(y.astype(bf16), w1_ref[...], preferred_element_type=f32) + b1_ref[...]
    hm = jnp.maximum(hm, 0.0).astype(bf16)
    ff = jnp.dot(hm, w2_ref[...], preferred_element_type=f32) + b2_ref[...]
    return _ln(y + ff, g2_ref[...], be2_ref[...])


def _tlayer_kernel(x_ref, *refs):
    (wqkv, bqkv, wo, bo, g1, be1, w1, b1, w2, b2, g2, be2, mask, o_ref) = refs
    x = x_ref[0].reshape(RT, D)
    z = _attn_ffn(x, wqkv, bqkv, wo, bo, g1, be1, w1, b1, w2, b2, g2, be2,
                  mask[...], None)
    o_ref[0] = z.reshape(T, GT, D)


def _slayer_kernel(x_ref, *refs):
    (wqkv, bqkv, wo, bo, g1, be1, w1, b1, w2, b2, g2, be2, rowmask, o_ref) = refs
    x = x_ref[0, 0]
    z = _attn_ffn(x, wqkv, bqkv, wo, bo, g1, be1, w1, b1, w2, b2, g2, be2,
                  None, rowmask[...])
    o_ref[0, 0] = z


def _head_kernel(x_ref, w_ref, b_ref, o_ref):
    acc = jnp.zeros((NP_, OUT), f32)
    for t in range(T):
        acc = acc + jnp.dot(x_ref[0, t].astype(bf16), w_ref[t],
                            preferred_element_type=f32)
    o_ref[0] = acc + b_ref[...]


def _prep_layer(p):
    def headpad_cols(w, b, scale=1.0):
        w3 = jnp.pad(w.reshape(D, H, HD) * scale, ((0, 0), (0, 0), (0, HP - HD)))
        b2 = jnp.pad(b.reshape(H, HD) * scale, ((0, 0), (0, HP - HD)))
        return w3.reshape(D, DH), b2.reshape(DH)

    wq, bq = headpad_cols(p["q"]["w"], p["q"]["b"], _SCALE)
    wk, bk = headpad_cols(p["k"]["w"], p["k"]["b"])
    wv, bv = headpad_cols(p["v"]["w"], p["v"]["b"])
    wqkv = jnp.concatenate([wq, wk, wv], axis=1).astype(bf16)
    bqkv = jnp.concatenate([bq, bk, bv]).reshape(1, 3 * DH)
    wo = jnp.pad(p["o"]["w"].reshape(H, HD, D),
                 ((0, 0), (0, HP - HD), (0, 0))).reshape(DH, D).astype(bf16)
    return [wqkv, bqkv, wo, p["o"]["b"].reshape(1, D),
            p["ln1g"].reshape(1, D), p["ln1b"].reshape(1, D),
            p["ff1"]["w"].astype(bf16), p["ff1"]["b"].reshape(1, FF),
            p["ff2"]["w"].astype(bf16), p["ff2"]["b"].reshape(1, D),
            p["ln2g"].reshape(1, D), p["ln2b"].reshape(1, D)]


def kernel(x, params):
    x2 = jnp.pad(x, ((0, 0), (0, 0), (0, NP_ - N), (0, 0)))
    adp = jnp.pad(params["adp"], ((0, 0), (0, NP_ - N), (0, 0)))
    tod_tab = params["tod_tab"]
    dow_tab = jnp.pad(params["dow_tab"], ((0, 1), (0, 0)))
    w_in = params["in_proj"]["w"]
    b_in = params["in_proj"]["b"].reshape(1, IN_EMB)
    mask_t = jnp.asarray(_MASK_T)
    rowmask = jnp.asarray(_ROWMASK)

    h = pl.pallas_call(
        _embed_kernel,
        grid=(BT,),
        in_specs=[pl.BlockSpec((1, 1, NP_, 3), lambda g: (g // T, g % T, 0, 0)),
                  _full_spec(tod_tab), _full_spec(dow_tab),
                  _full_spec(w_in), _full_spec(b_in),
                  pl.BlockSpec((1, NP_, ADP_EMB), lambda g: (g % T, 0, 0))],
        out_specs=pl.BlockSpec((1, 1, NP_, D), lambda g: (g // T, g % T, 0, 0)),
        out_shape=jax.ShapeDtypeStruct((B, T, NP_, D), f32),
    )(x2, tod_tab, dow_tab, w_in, b_in, adp)

    for p in params["layers_t"]:
        wts = _prep_layer(p)
        h = pl.pallas_call(
            _tlayer_kernel,
            grid=(B * NGT,),
            in_specs=[pl.BlockSpec((1, T, GT, D), lambda g: (g // NGT, 0, g % NGT, 0))]
            + [_full_spec(w) for w in wts] + [_full_spec(mask_t)],
            out_specs=pl.BlockSpec((1, T, GT, D), lambda g: (g // NGT, 0, g % NGT, 0)),
            out_shape=jax.ShapeDtypeStruct((B, T, NP_, D), f32),
        )(h, *wts, mask_t)

    for p in params["layers_s"]:
        wts = _prep_layer(p)
        h = pl.pallas_call(
            _slayer_kernel,
            grid=(BT,),
            in_specs=[pl.BlockSpec((1, 1, NP_, D), lambda g: (g // T, g % T, 0, 0))]
            + [_full_spec(w) for w in wts] + [_full_spec(rowmask)],
            out_specs=pl.BlockSpec((1, 1, NP_, D), lambda g: (g // T, g % T, 0, 0)),
            out_shape=jax.ShapeDtypeStruct((B, T, NP_, D), f32),
        )(h, *wts, rowmask)

    w_out = params["out_proj"]["w"].reshape(T, D, OUT).astype(bf16)
    b_out = params["out_proj"]["b"].reshape(1, OUT)
    o = pl.pallas_call(
        _head_kernel,
        grid=(B,),
        in_specs=[pl.BlockSpec((1, T, NP_, D), lambda g: (g, 0, 0, 0)),
                  _full_spec(w_out), _full_spec(b_out)],
        out_specs=pl.BlockSpec((1, NP_, OUT), lambda g: (g, 0, 0)),
        out_shape=jax.ShapeDtypeStruct((B, NP_, OUT), f32),
    )(h, w_out, b_out)
    return o[:, :N].transpose(0, 2, 1)[..., None]
